# Initial kernel scaffold; baseline (speedup 1.0000x reference)
import jax
import jax.numpy as jnp
from jax.experimental import pallas as pl


def kernel(queries, keys, values, attn_mask):
    raise NotImplementedError("write your pallas kernel here")


# speedup vs baseline: 3.5877x; 3.5877x over previous
import functools
from math import sqrt

import jax
import jax.numpy as jnp
from jax.experimental import pallas as pl
from jax.experimental.pallas import tpu as pltpu

B, L, DIM, H, D = 2, 2048, 2, 12, 64
U = 24
NCHUNK = 16
NEG = -3.0e38


def _slice_bdh(s):
    return s // (DIM * H), (s // H) % DIM, s % H


def _kernel_body(q_ref, k_ref, v_ref, lo_ref, hi_ref, out_ref, m_ref, sel_ref):
    f32 = jnp.float32
    K_val = k_ref[0, :, :]
    V_val = v_ref[0, :, :]

    col = jax.lax.broadcasted_iota(jnp.int32, (128, 128), 1)
    col_valid = col < U

    def chunk_body(c, _):
        qc = q_ref[0, pl.ds(c * 128, 128), :]
        s_c = jax.lax.dot_general(
            qc, K_val, (((1,), (1,)), ((), ())),
            precision=jax.lax.Precision.HIGHEST,
            preferred_element_type=f32)
        lo_c = lo_ref[pl.ds(c * 128, 128), :]
        hi_c = hi_ref[pl.ds(c * 128, 128), :]
        g = jnp.zeros((128, 128), f32)
        for t in range(NCHUNK):
            sub = s_c[:, t * 128:(t + 1) * 128]
            gt = jnp.take_along_axis(sub, lo_c, axis=1)
            g = jnp.where(hi_c == t, gt, g)
        gmax = jnp.max(jnp.where(col_valid, g, NEG), axis=1)
        gsum = jnp.sum(g, axis=1)
        m_ref[c, :] = gmax - gsum / float(L)
        return 0

    jax.lax.fori_loop(0, NCHUNK, chunk_body, 0, unroll=False)

    flat = (jax.lax.broadcasted_iota(jnp.int32, (NCHUNK, 128), 0) * 128
            + jax.lax.broadcasted_iota(jnp.int32, (NCHUNK, 128), 1))

    def topk_body(t, m_val):
        mx = jnp.max(m_val)
        cand = jnp.where(m_val == mx, flat, jnp.int32(2 * L))
        i = jnp.min(cand)
        sel_ref[t] = i
        return jnp.where(flat == i, NEG, m_val)

    jax.lax.fori_loop(0, U, topk_body, m_ref[:, :], unroll=False)

    rows = []
    for t in range(U):
        qt = sel_ref[t]
        rows.append(q_ref[0, pl.ds(qt, 1), :])
    q_sel = jnp.concatenate(rows, axis=0)
    scores = jax.lax.dot_general(
        q_sel, K_val, (((1,), (1,)), ((), ())),
        precision=jax.lax.Precision.HIGHEST,
        preferred_element_type=f32) * (1.0 / sqrt(D))
    smax = jnp.max(scores, axis=1, keepdims=True)
    unnorm = jnp.exp(scores - smax)
    attn = unnorm / jnp.sum(unnorm, axis=1, keepdims=True)
    out24 = jax.lax.dot_general(
        attn, V_val, (((1,), (0,)), ((), ())),
        precision=jax.lax.Precision.HIGHEST,
        preferred_element_type=f32)

    vmean = jnp.mean(V_val, axis=0, keepdims=True)
    out_ref[0, :, :] = jnp.broadcast_to(vmean, (L, D))
    for t in range(U):
        qt = sel_ref[t]
        out_ref[0, pl.ds(qt, 1), :] = out24[t:t + 1, :]


@jax.jit
def kernel(queries, keys, values, attn_mask):
    del attn_mask
    idx = jax.random.randint(jax.random.key(42), (L, U), 0, L)
    lo = jnp.concatenate(
        [idx % 128, jnp.zeros((L, 128 - U), jnp.int32)], axis=1)
    hi = jnp.concatenate(
        [idx // 128, jnp.full((L, 128 - U), -1, jnp.int32)], axis=1)

    def to_slices(x):
        return jnp.transpose(x, (0, 2, 3, 1, 4)).reshape(B * DIM * H, L, D)

    qkv_spec = pl.BlockSpec((1, L, D), lambda s: (s, 0, 0))
    idx_spec = pl.BlockSpec((L, 128), lambda s: (0, 0))

    out = pl.pallas_call(
        _kernel_body,
        grid=(B * DIM * H,),
        in_specs=[qkv_spec, qkv_spec, qkv_spec, idx_spec, idx_spec],
        out_specs=qkv_spec,
        out_shape=jax.ShapeDtypeStruct((B * DIM * H, L, D), jnp.float32),
        scratch_shapes=[
            pltpu.VMEM((NCHUNK, 128), jnp.float32),
            pltpu.SMEM((U,), jnp.int32),
        ],
    )(to_slices(queries), to_slices(keys), to_slices(values), lo, hi)
    return out.reshape(B, DIM, H, L, D)
